# baseline (speedup 1.0000x reference)
import functools

import jax
import jax.numpy as jnp
from jax import lax
from jax.experimental import pallas as pl
from jax.experimental.pallas import tpu as pltpu
from jax.experimental.pallas import tpu_sc as plsc

E = 64
D = 768
CAPACITY = 2
S = 8192
K_TOTAL = S * CAPACITY
TILE = 128
NT = 192
PADN = NT * TILE
NW = 32
L = 16
GC = 64
CHT = 1024
SLICE = PADN // 16


def _read_lane_i32(vecref, idx):
    base = pl.multiple_of(lax.div(idx, L) * L, L)
    win = vecref[pl.ds(base, L)]
    lane = idx - base
    return jnp.sum(jnp.where(lax.iota(jnp.int32, L) == lane, win, 0))


def _gates_kernel(x_ref, gw_ref, gt_ref, offp_ref, te_ref):
    logits = lax.dot_general(
        gw_ref[...], x_ref[...], (((1,), (1,)), ((), ())),
        preferred_element_type=jnp.float32)
    m = jnp.max(logits, axis=0, keepdims=True)
    ex = jnp.exp(logits - m)
    scores = ex / jnp.sum(ex, axis=0, keepdims=True)

    bits = lax.bitcast_convert_type(scores, jnp.int32)

    def body(_, lohi):
        lo, hi = lohi
        mid = lax.div(lo + hi + 1, jnp.int32(2))
        cnt = jnp.sum((bits >= mid).astype(jnp.int32))
        take = cnt >= K_TOTAL
        return jnp.where(take, mid, lo), jnp.where(take, hi, mid - 1)

    lo, _ = lax.fori_loop(0, 30, body, (jnp.int32(0), jnp.int32(0x3F800000)))
    sel = bits >= lo
    gt_ref[...] = jnp.where(sel, scores, 0.0)

    ones_row = jnp.ones((1, S), jnp.float32)
    cnt_row = lax.dot_general(
        ones_row, sel.astype(jnp.float32), (((1,), (1,)), ((), ())),
        preferred_element_type=jnp.float32)
    padded = jnp.floor((cnt_row + (TILE - 1)) / TILE) * TILE
    r = lax.broadcasted_iota(jnp.int32, (E, E), 0)
    c = lax.broadcasted_iota(jnp.int32, (E, E), 1)
    upper = (r <= c).astype(jnp.float32)
    incl = jnp.dot(padded, upper, preferred_element_type=jnp.float32)
    incl = jnp.minimum(incl, float(PADN))
    excl = jnp.minimum(incl - padded, float(PADN))
    total = jnp.max(incl)

    lane = lax.broadcasted_iota(jnp.int32, (1, 2 * E), 1)
    both = jnp.concatenate([excl, incl], axis=1)
    offp_ref[...] = jnp.where(lane < E, both, total).astype(jnp.int32)

    tv = lax.broadcasted_iota(jnp.int32, (256, 1), 0).astype(jnp.float32) * TILE
    tef = jnp.sum((incl <= tv).astype(jnp.float32), axis=1, keepdims=True)
    te_ref[...] = jnp.minimum(tef, float(E - 1)).astype(jnp.int32)


def _dispatch_kernel(gt_hbm, offp_hbm, x_hbm, tok_hbm, gate_hbm, xg_hbm,
                     row_v, tokbuf, gatebuf, offp_v, cntbuf, zt_v, zg_v,
                     rows_a, rows_b, sem_a, sem_b):
    wid = lax.axis_index("s") * 2 + lax.axis_index("c")
    pltpu.sync_copy(offp_hbm, offp_v)

    zero16i = jnp.zeros((L,), jnp.int32)
    zero16f = jnp.zeros((L,), jnp.float32)
    sent16 = jnp.full((L,), S, jnp.int32)
    iota16 = lax.iota(jnp.int32, L)
    for zi in range(TILE // L):
        zt_v[pl.ds(zi * L, L)] = sent16
        zg_v[pl.ds(zi * L, L)] = zero16f

    def do_expert(e):
        pltpu.sync_copy(gt_hbm.at[e], row_v)

        def zbody(i, _):
            o = pl.multiple_of(i * L, L)
            tokbuf[pl.ds(o, L)] = sent16
            gatebuf[pl.ds(o, L)] = zero16f
            return 0
        lax.fori_loop(0, (S + TILE) // L, zbody, 0)

        def sbody(i, cnt):
            v = row_v[pl.ds(pl.multiple_of(i * L, L), L)]
            msk = v > 0.0
            pc = plsc.cumsum(msk.astype(jnp.int32))
            pos = cnt + pc - 1
            plsc.store_scatter(tokbuf, [pos], iota16 + i * L, mask=msk)
            plsc.store_scatter(gatebuf, [pos], v, mask=msk)
            return cnt + pc[L - 1]
        cnt = lax.fori_loop(0, S // L, sbody, zero16i)
        cntbuf[...] = cnt
        c = cntbuf[...][0]

        excl = _read_lane_i32(offp_v, e)
        nch = jnp.minimum(lax.div(c + TILE - 1, TILE),
                          lax.div(PADN - excl, TILE))

        def wbody(j, _):
            jo = pl.multiple_of(j * TILE, TILE)
            go = pl.multiple_of(excl + j * TILE, TILE)
            pltpu.sync_copy(tokbuf.at[pl.ds(jo, TILE)],
                            tok_hbm.at[pl.ds(go, TILE)])
            pltpu.sync_copy(gatebuf.at[pl.ds(jo, TILE)],
                            gate_hbm.at[pl.ds(go, TILE)])
            return 0
        lax.fori_loop(0, nch, wbody, 0)

        ngc = nch * (TILE // GC)

        def gbody(j2, _):
            j = j2 * 2
            c0 = j < ngc
            c1 = j + 1 < ngc
            jo0 = pl.multiple_of(j * GC, GC)
            jo1 = pl.multiple_of(j * GC + GC, GC)
            go0 = pl.multiple_of(excl + j * GC, GC)
            go1 = pl.multiple_of(excl + j * GC + GC, GC)

            @pl.when(c0)
            def _():
                pltpu.async_copy(x_hbm.at[tokbuf.at[pl.ds(jo0, GC)]],
                                 rows_a, sem_a)

            @pl.when(c1)
            def _():
                pltpu.async_copy(x_hbm.at[tokbuf.at[pl.ds(jo1, GC)]],
                                 rows_b, sem_b)

            @pl.when(c0)
            def _():
                pltpu.make_async_copy(x_hbm.at[tokbuf.at[pl.ds(jo0, GC)]],
                                      rows_a, sem_a).wait()
                pltpu.sync_copy(rows_a, xg_hbm.at[pl.ds(go0, GC)])

            @pl.when(c1)
            def _():
                pltpu.make_async_copy(x_hbm.at[tokbuf.at[pl.ds(jo1, GC)]],
                                      rows_b, sem_b).wait()
                pltpu.sync_copy(rows_b, xg_hbm.at[pl.ds(go1, GC)])
            return 0
        lax.fori_loop(0, lax.div(ngc + 1, 2), gbody, 0)

    do_expert(wid * 2)
    do_expert(wid * 2 + 1)

    @pl.when(wid == NW - 1)
    def _tail():
        total = offp_v[pl.ds(E, L)][0]

        def tbody(j, _):
            off = pl.multiple_of(total + j * TILE, TILE)

            @pl.when(off < PADN)
            def _():
                pltpu.sync_copy(zt_v, tok_hbm.at[pl.ds(off, TILE)])
                pltpu.sync_copy(zg_v, gate_hbm.at[pl.ds(off, TILE)])
            return 0
        lax.fori_loop(0, NT, tbody, 0)


def _gmm_kernel(te_ref, xg_ref, w_ref, b_ref, gl_ref, yg_ref):
    y = lax.dot_general(
        xg_ref[...], w_ref[0], (((1,), (1,)), ((), ())),
        preferred_element_type=jnp.float32)
    g = gl_ref[0, 0]
    gc = g[:, None]
    yg_ref[...] = jnp.where(gc > 0.0, (y + b_ref[0]) * gc, 0.0)


L1CAP = 8192
L2CAP = 2048
TB = 32
NP = (S // NW) // TB


def _combine_kernel(yg_hbm, tok_hbm, out_hbm,
                    tokch, l1src, l1loc, l2src, l2loc, cntbuf,
                    acc, rows_a, rows_b, sem_a, sem_b):
    wid = lax.axis_index("s") * 2 + lax.axis_index("c")
    mybase = wid * (S // NW)
    zero16i = jnp.zeros((L,), jnp.int32)
    iota16 = lax.iota(jnp.int32, L)

    pltpu.sync_copy(tok_hbm, tokch)

    def scan1(i, cnt):
        o = pl.multiple_of(i * L, L)
        tv = tokch[pl.ds(o, L)]
        msk = (tv >= mybase) & (tv < mybase + S // NW)
        pc = plsc.cumsum(msk.astype(jnp.int32))
        pos = cnt + pc - 1
        msk = msk & (pos < L1CAP)
        plsc.store_scatter(l1src, [pos], i * L + iota16, mask=msk)
        plsc.store_scatter(l1loc, [pos], tv - mybase, mask=msk)
        return cnt + pc[L - 1]
    cnt1 = lax.fori_loop(0, PADN // L, scan1, zero16i)
    cntbuf[...] = jnp.minimum(cnt1, L1CAP)
    c1 = cntbuf[...][0]
    n1 = lax.div(c1 + L - 1, L)

    for p in range(NP):
        def zacc(i, _):
            for q in range(D // L):
                acc[i, pl.ds(q * L, L)] = jnp.zeros((L,), jnp.float32)
            return 0
        lax.fori_loop(0, TB, zacc, 0)

        def zl2(i, _):
            for q in range(TB // L):
                l2src[i, pl.ds(q * L, L)] = zero16i
            return 0
        lax.fori_loop(0, L2CAP // TB, zl2, 0)

        lo = p * TB

        def scan2(i, cnt):
            o = pl.multiple_of(i * L, L)
            lv = l1loc[pl.ds(o, L)]
            sv = l1src[pl.ds(o, L)]
            valid = (i * L + iota16) < c1
            msk = valid & (lv >= lo) & (lv < lo + TB)
            pc = plsc.cumsum(msk.astype(jnp.int32))
            pos = cnt + pc - 1
            msk = msk & (pos < L2CAP)
            plsc.store_scatter(l2src, [lax.div(pos, TB), lax.rem(pos, TB)],
                               sv, mask=msk)
            plsc.store_scatter(l2loc, [pos], lv - lo, mask=msk)
            return cnt + pc[L - 1]
        cnt2 = lax.fori_loop(0, n1, scan2, zero16i)
        cntbuf[...] = jnp.minimum(cnt2, L2CAP)
        c2 = cntbuf[...][0]
        nb = lax.div(c2 + TB - 1, TB)

        def addrows(rbuf, b):
            nr = jnp.minimum(c2 - b * TB, TB)

            def rbody(j, _):
                lj = _read_lane_i32(l2loc, b * TB + j)
                for q in range(D // L):
                    sl = pl.ds(q * L, L)
                    acc[lj, sl] = acc[lj, sl] + rbuf[j, sl]
                return 0
            lax.fori_loop(0, nr, rbody, 0)

        def bbody(b2, _):
            b = b2 * 2
            c0 = b < nb
            c1b = b + 1 < nb

            @pl.when(c0)
            def _():
                pltpu.async_copy(yg_hbm.at[l2src.at[b]], rows_a, sem_a)

            @pl.when(c1b)
            def _():
                pltpu.async_copy(yg_hbm.at[l2src.at[b + 1]], rows_b, sem_b)

            @pl.when(c0)
            def _():
                pltpu.make_async_copy(yg_hbm.at[l2src.at[b]],
                                      rows_a, sem_a).wait()
                addrows(rows_a, b)

            @pl.when(c1b)
            def _():
                pltpu.make_async_copy(yg_hbm.at[l2src.at[b + 1]],
                                      rows_b, sem_b).wait()
                addrows(rows_b, b + 1)
            return 0
        lax.fori_loop(0, lax.div(nb + 1, 2), bbody, 0)

        pltpu.sync_copy(acc, out_hbm.at[pl.ds(mybase + lo, TB)])


def kernel(x, gate_weight, expert_w, expert_b):
    Bb, s, _ = x.shape
    x_flat = x.reshape(-1, D)

    gatesT, offp, te = pl.pallas_call(
        _gates_kernel,
        out_shape=(
            jax.ShapeDtypeStruct((E, S), jnp.float32),
            jax.ShapeDtypeStruct((1, 2 * E), jnp.int32),
            jax.ShapeDtypeStruct((256, 1), jnp.int32),
        ),
    )(x_flat, gate_weight)

    mesh = plsc.VectorSubcoreMesh(core_axis_name="c", subcore_axis_name="s")
    sc_params = pltpu.CompilerParams(needs_layout_passes=False,
                                     use_tc_tiling_on_sc=False)

    xpad = jnp.concatenate([x_flat, jnp.zeros((8, D), jnp.float32)], axis=0)

    tok, gate, xg = pl.kernel(
        _dispatch_kernel,
        out_type=(
            jax.ShapeDtypeStruct((PADN,), jnp.int32),
            jax.ShapeDtypeStruct((PADN,), jnp.float32),
            jax.ShapeDtypeStruct((PADN, D), jnp.float32),
        ),
        mesh=mesh,
        compiler_params=sc_params,
        scratch_types=[
            pltpu.VMEM((S,), jnp.float32),
            pltpu.VMEM((S + TILE,), jnp.int32),
            pltpu.VMEM((S + TILE,), jnp.float32),
            pltpu.VMEM((2 * E,), jnp.int32),
            pltpu.VMEM((L,), jnp.int32),
            pltpu.VMEM((TILE,), jnp.int32),
            pltpu.VMEM((TILE,), jnp.float32),
            pltpu.VMEM((GC, D), jnp.float32),
            pltpu.VMEM((GC, D), jnp.float32),
            pltpu.SemaphoreType.DMA,
            pltpu.SemaphoreType.DMA,
        ],
    )(gatesT, offp.reshape(2 * E), xpad)

    yg = pl.pallas_call(
        _gmm_kernel,
        grid_spec=pltpu.PrefetchScalarGridSpec(
            num_scalar_prefetch=1,
            grid=(NT,),
            in_specs=[
                pl.BlockSpec((TILE, D), lambda i, te_r: (i, 0)),
                pl.BlockSpec((1, D, D), lambda i, te_r: (te_r[i], 0, 0)),
                pl.BlockSpec((1, 1, D), lambda i, te_r: (te_r[i], 0, 0)),
                pl.BlockSpec((1, 1, TILE), lambda i, te_r: (i, 0, 0)),
            ],
            out_specs=pl.BlockSpec((TILE, D), lambda i, te_r: (i, 0)),
        ),
        out_shape=jax.ShapeDtypeStruct((PADN, D), jnp.float32),
        compiler_params=pltpu.CompilerParams(
            dimension_semantics=("arbitrary",),
        ),
    )(te.reshape(256), xg, expert_w, expert_b.reshape(E, 1, D),
      gate.reshape(NT, 1, TILE))

    out = pl.kernel(
        _combine_kernel,
        out_type=jax.ShapeDtypeStruct((S, D), jnp.float32),
        mesh=mesh,
        compiler_params=sc_params,
        scratch_types=[
            pltpu.VMEM((PADN,), jnp.int32),
            pltpu.VMEM((L1CAP,), jnp.int32),
            pltpu.VMEM((L1CAP,), jnp.int32),
            pltpu.VMEM((L2CAP // TB, TB), jnp.int32),
            pltpu.VMEM((L2CAP + L,), jnp.int32),
            pltpu.VMEM((L,), jnp.int32),
            pltpu.VMEM((TB, D), jnp.float32),
            pltpu.VMEM((TB, D), jnp.float32),
            pltpu.VMEM((TB, D), jnp.float32),
            pltpu.SemaphoreType.DMA,
            pltpu.SemaphoreType.DMA,
        ],
    )(yg, tok)

    return out.reshape(Bb, s, D)


# speedup vs baseline: 1.2530x; 1.2530x over previous
import functools

import jax
import jax.numpy as jnp
from jax import lax
from jax.experimental import pallas as pl
from jax.experimental.pallas import tpu as pltpu
from jax.experimental.pallas import tpu_sc as plsc

E = 64
D = 768
CAPACITY = 2
S = 8192
K_TOTAL = S * CAPACITY
TILE = 128
NT = 192
PADN = NT * TILE
NW = 32
L = 16
GC = 64
CHT = 1024
SLICE = PADN // 16


def _read_lane_i32(vecref, idx):
    base = pl.multiple_of(lax.div(idx, L) * L, L)
    win = vecref[pl.ds(base, L)]
    lane = idx - base
    return jnp.sum(jnp.where(lax.iota(jnp.int32, L) == lane, win, 0))


def _gates_kernel(x_ref, gw_ref, gt_ref, offp_ref, te_ref):
    logits = lax.dot_general(
        gw_ref[...], x_ref[...], (((1,), (1,)), ((), ())),
        preferred_element_type=jnp.float32)
    m = jnp.max(logits, axis=0, keepdims=True)
    ex = jnp.exp(logits - m)
    scores = ex / jnp.sum(ex, axis=0, keepdims=True)

    bits = lax.bitcast_convert_type(scores, jnp.int32)

    def body(_, lohi):
        lo, hi = lohi
        mid = lax.div(lo + hi + 1, jnp.int32(2))
        cnt = jnp.sum((bits >= mid).astype(jnp.int32))
        take = cnt >= K_TOTAL
        return jnp.where(take, mid, lo), jnp.where(take, hi, mid - 1)

    lo, _ = lax.fori_loop(0, 30, body, (jnp.int32(0), jnp.int32(0x3F800000)))
    sel = bits >= lo
    gt_ref[...] = jnp.where(sel, scores, 0.0)

    ones_row = jnp.ones((1, S), jnp.float32)
    cnt_row = lax.dot_general(
        ones_row, sel.astype(jnp.float32), (((1,), (1,)), ((), ())),
        preferred_element_type=jnp.float32)
    padded = jnp.floor((cnt_row + (TILE - 1)) / TILE) * TILE
    r = lax.broadcasted_iota(jnp.int32, (E, E), 0)
    c = lax.broadcasted_iota(jnp.int32, (E, E), 1)
    upper = (r <= c).astype(jnp.float32)
    incl = jnp.dot(padded, upper, preferred_element_type=jnp.float32)
    incl = jnp.minimum(incl, float(PADN))
    excl = jnp.minimum(incl - padded, float(PADN))
    total = jnp.max(incl)

    lane = lax.broadcasted_iota(jnp.int32, (1, 2 * E), 1)
    both = jnp.concatenate([excl, incl], axis=1)
    offp_ref[...] = jnp.where(lane < E, both, total).astype(jnp.int32)

    tv = lax.broadcasted_iota(jnp.int32, (256, 1), 0).astype(jnp.float32) * TILE
    tef = jnp.sum((incl <= tv).astype(jnp.float32), axis=1, keepdims=True)
    te_ref[...] = jnp.minimum(tef, float(E - 1)).astype(jnp.int32)


def _dispatch_kernel(gt_hbm, offp_hbm, x_hbm, tok_hbm, gate_hbm, xg_hbm,
                     row_v, tokbuf, gatebuf, offp_v, cntbuf, zt_v, zg_v,
                     rows_a, rows_b, sem_a, sem_b):
    wid = lax.axis_index("s") * 2 + lax.axis_index("c")
    pltpu.sync_copy(offp_hbm, offp_v)

    zero16i = jnp.zeros((L,), jnp.int32)
    zero16f = jnp.zeros((L,), jnp.float32)
    sent16 = jnp.full((L,), S, jnp.int32)
    iota16 = lax.iota(jnp.int32, L)
    for zi in range(TILE // L):
        zt_v[pl.ds(zi * L, L)] = sent16
        zg_v[pl.ds(zi * L, L)] = zero16f

    def do_expert(e):
        pltpu.sync_copy(gt_hbm.at[e], row_v)

        def zbody(i, _):
            o = pl.multiple_of(i * L, L)
            tokbuf[pl.ds(o, L)] = sent16
            gatebuf[pl.ds(o, L)] = zero16f
            return 0
        lax.fori_loop(0, (S + TILE) // L, zbody, 0)

        def sbody(i, cnt):
            v = row_v[pl.ds(pl.multiple_of(i * L, L), L)]
            msk = v > 0.0
            pc = plsc.cumsum(msk.astype(jnp.int32))
            pos = cnt + pc - 1
            plsc.store_scatter(tokbuf, [pos], iota16 + i * L, mask=msk)
            plsc.store_scatter(gatebuf, [pos], v, mask=msk)
            return cnt + pc[L - 1]
        cnt = lax.fori_loop(0, S // L, sbody, zero16i)
        cntbuf[...] = cnt
        c = cntbuf[...][0]

        excl = _read_lane_i32(offp_v, e)
        nch = jnp.minimum(lax.div(c + TILE - 1, TILE),
                          lax.div(PADN - excl, TILE))

        def wbody(j, _):
            jo = pl.multiple_of(j * TILE, TILE)
            go = pl.multiple_of(excl + j * TILE, TILE)
            pltpu.sync_copy(tokbuf.at[pl.ds(jo, TILE)],
                            tok_hbm.at[pl.ds(go, TILE)])
            pltpu.sync_copy(gatebuf.at[pl.ds(jo, TILE)],
                            gate_hbm.at[pl.ds(go, TILE)])
            return 0
        lax.fori_loop(0, nch, wbody, 0)

        ngc = nch * (TILE // GC)

        def gbody(j2, _):
            j = j2 * 2
            c0 = j < ngc
            c1 = j + 1 < ngc
            jo0 = pl.multiple_of(j * GC, GC)
            jo1 = pl.multiple_of(j * GC + GC, GC)
            go0 = pl.multiple_of(excl + j * GC, GC)
            go1 = pl.multiple_of(excl + j * GC + GC, GC)

            @pl.when(c0)
            def _():
                pltpu.async_copy(x_hbm.at[tokbuf.at[pl.ds(jo0, GC)]],
                                 rows_a, sem_a)

            @pl.when(c1)
            def _():
                pltpu.async_copy(x_hbm.at[tokbuf.at[pl.ds(jo1, GC)]],
                                 rows_b, sem_b)

            @pl.when(c0)
            def _():
                pltpu.make_async_copy(x_hbm.at[tokbuf.at[pl.ds(jo0, GC)]],
                                      rows_a, sem_a).wait()
                pltpu.sync_copy(rows_a, xg_hbm.at[pl.ds(go0, GC)])

            @pl.when(c1)
            def _():
                pltpu.make_async_copy(x_hbm.at[tokbuf.at[pl.ds(jo1, GC)]],
                                      rows_b, sem_b).wait()
                pltpu.sync_copy(rows_b, xg_hbm.at[pl.ds(go1, GC)])
            return 0
        lax.fori_loop(0, lax.div(ngc + 1, 2), gbody, 0)

    do_expert(wid * 2)
    do_expert(wid * 2 + 1)

    @pl.when(wid == NW - 1)
    def _tail():
        total = offp_v[pl.ds(E, L)][0]

        def tbody(j, _):
            off = pl.multiple_of(total + j * TILE, TILE)

            @pl.when(off < PADN)
            def _():
                pltpu.sync_copy(zt_v, tok_hbm.at[pl.ds(off, TILE)])
                pltpu.sync_copy(zg_v, gate_hbm.at[pl.ds(off, TILE)])
            return 0
        lax.fori_loop(0, NT, tbody, 0)


def _gmm_kernel(te_ref, xg_ref, w_ref, b_ref, gl_ref, yg_ref):
    y = lax.dot_general(
        xg_ref[...], w_ref[0], (((1,), (1,)), ((), ())),
        preferred_element_type=jnp.float32)
    g = gl_ref[0, 0]
    gc = g[:, None]
    yg_ref[...] = jnp.where(gc > 0.0, (y + b_ref[0]) * gc, 0.0)


L1CAP = 8192
L2CAP = 2048
TB = 32
NP = (S // NW) // TB


def _combine_kernel(yg_hbm, tok_hbm, out_hbm,
                    tokch, l1src, l1loc, l2src, l2loc, cntbuf,
                    acc, rows_a, rows_b, sem_a, sem_b):
    wid = lax.axis_index("s") * 2 + lax.axis_index("c")
    mybase = wid * (S // NW)
    zero16i = jnp.zeros((L,), jnp.int32)
    iota16 = lax.iota(jnp.int32, L)

    pltpu.sync_copy(tok_hbm, tokch)

    def scan1(i, cnt):
        o = pl.multiple_of(i * L, L)
        tv = tokch[pl.ds(o, L)]
        msk = (tv >= mybase) & (tv < mybase + S // NW)
        pc = plsc.cumsum(msk.astype(jnp.int32))
        pos = cnt + pc - 1
        msk = msk & (pos < L1CAP)
        plsc.store_scatter(l1src, [pos], i * L + iota16, mask=msk)
        plsc.store_scatter(l1loc, [pos], tv - mybase, mask=msk)
        return cnt + pc[L - 1]
    cnt1 = lax.fori_loop(0, PADN // L, scan1, zero16i)
    cntbuf[...] = jnp.minimum(cnt1, L1CAP)
    c1 = cntbuf[...][0]
    n1 = lax.div(c1 + L - 1, L)

    for p in range(NP):
        def zacc(i, _):
            for q in range(D // L):
                acc[i, pl.ds(q * L, L)] = jnp.zeros((L,), jnp.float32)
            return 0
        lax.fori_loop(0, TB, zacc, 0)

        def zl2(i, _):
            for q in range(TB // L):
                l2src[i, pl.ds(q * L, L)] = zero16i
            return 0
        lax.fori_loop(0, L2CAP // TB, zl2, 0)

        lo = p * TB

        def scan2(i, cnt):
            o = pl.multiple_of(i * L, L)
            lv = l1loc[pl.ds(o, L)]
            sv = l1src[pl.ds(o, L)]
            valid = (i * L + iota16) < c1
            msk = valid & (lv >= lo) & (lv < lo + TB)
            pc = plsc.cumsum(msk.astype(jnp.int32))
            pos = cnt + pc - 1
            msk = msk & (pos < L2CAP)
            plsc.store_scatter(l2src, [lax.div(pos, TB), lax.rem(pos, TB)],
                               sv, mask=msk)
            plsc.store_scatter(l2loc, [pos], lv - lo, mask=msk)
            return cnt + pc[L - 1]
        cnt2 = lax.fori_loop(0, n1, scan2, zero16i)
        cntbuf[...] = jnp.minimum(cnt2, L2CAP)
        c2 = cntbuf[...][0]
        nb = lax.div(c2 + TB - 1, TB)

        def addrows(rbuf, b):
            nr = jnp.minimum(c2 - b * TB, TB)

            def rbody(j, _):
                lj = _read_lane_i32(l2loc, b * TB + j)
                for q in range(D // L):
                    sl = pl.ds(q * L, L)
                    acc[lj, sl] = acc[lj, sl] + rbuf[j, sl]
                return 0
            lax.fori_loop(0, nr, rbody, 0)

        def bbody(b2, _):
            b = b2 * 2
            c0 = b < nb
            c1b = b + 1 < nb

            @pl.when(c0)
            def _():
                pltpu.async_copy(yg_hbm.at[l2src.at[b]], rows_a, sem_a)

            @pl.when(c1b)
            def _():
                pltpu.async_copy(yg_hbm.at[l2src.at[b + 1]], rows_b, sem_b)

            @pl.when(c0)
            def _():
                pltpu.make_async_copy(yg_hbm.at[l2src.at[b]],
                                      rows_a, sem_a).wait()
                addrows(rows_a, b)

            @pl.when(c1b)
            def _():
                pltpu.make_async_copy(yg_hbm.at[l2src.at[b + 1]],
                                      rows_b, sem_b).wait()
                addrows(rows_b, b + 1)
            return 0
        lax.fori_loop(0, lax.div(nb + 1, 2), bbody, 0)

        pltpu.sync_copy(acc, out_hbm.at[pl.ds(mybase + lo, TB)])


def kernel(x, gate_weight, expert_w, expert_b):
    Bb, s, _ = x.shape
    x_flat = x.reshape(-1, D)

    gatesT, offp, te = pl.pallas_call(
        _gates_kernel,
        out_shape=(
            jax.ShapeDtypeStruct((E, S), jnp.float32),
            jax.ShapeDtypeStruct((1, 2 * E), jnp.int32),
            jax.ShapeDtypeStruct((256, 1), jnp.int32),
        ),
    )(x_flat, gate_weight)

    mesh = plsc.VectorSubcoreMesh(core_axis_name="c", subcore_axis_name="s")
    sc_params = pltpu.CompilerParams(needs_layout_passes=False)

    xpad = jnp.concatenate([x_flat, jnp.zeros((8, D), jnp.float32)], axis=0)

    tok, gate, xg = pl.kernel(
        _dispatch_kernel,
        out_type=(
            jax.ShapeDtypeStruct((PADN,), jnp.int32),
            jax.ShapeDtypeStruct((PADN,), jnp.float32),
            jax.ShapeDtypeStruct((PADN, D), jnp.float32),
        ),
        mesh=mesh,
        compiler_params=sc_params,
        scratch_types=[
            pltpu.VMEM((S,), jnp.float32),
            pltpu.VMEM((S + TILE,), jnp.int32),
            pltpu.VMEM((S + TILE,), jnp.float32),
            pltpu.VMEM((2 * E,), jnp.int32),
            pltpu.VMEM((L,), jnp.int32),
            pltpu.VMEM((TILE,), jnp.int32),
            pltpu.VMEM((TILE,), jnp.float32),
            pltpu.VMEM((GC, D), jnp.float32),
            pltpu.VMEM((GC, D), jnp.float32),
            pltpu.SemaphoreType.DMA,
            pltpu.SemaphoreType.DMA,
        ],
    )(gatesT, offp.reshape(2 * E), xpad)

    yg = pl.pallas_call(
        _gmm_kernel,
        grid_spec=pltpu.PrefetchScalarGridSpec(
            num_scalar_prefetch=1,
            grid=(NT,),
            in_specs=[
                pl.BlockSpec((TILE, D), lambda i, te_r: (i, 0)),
                pl.BlockSpec((1, D, D), lambda i, te_r: (te_r[i], 0, 0)),
                pl.BlockSpec((1, 1, D), lambda i, te_r: (te_r[i], 0, 0)),
                pl.BlockSpec((1, 1, TILE), lambda i, te_r: (i, 0, 0)),
            ],
            out_specs=pl.BlockSpec((TILE, D), lambda i, te_r: (i, 0)),
        ),
        out_shape=jax.ShapeDtypeStruct((PADN, D), jnp.float32),
        compiler_params=pltpu.CompilerParams(
            dimension_semantics=("arbitrary",),
        ),
    )(te.reshape(256), xg, expert_w, expert_b.reshape(E, 1, D),
      gate.reshape(NT, 1, TILE))

    out = pl.kernel(
        _combine_kernel,
        out_type=jax.ShapeDtypeStruct((S, D), jnp.float32),
        mesh=mesh,
        compiler_params=sc_params,
        scratch_types=[
            pltpu.VMEM((PADN,), jnp.int32),
            pltpu.VMEM((L1CAP,), jnp.int32),
            pltpu.VMEM((L1CAP,), jnp.int32),
            pltpu.VMEM((L2CAP // TB, TB), jnp.int32),
            pltpu.VMEM((L2CAP + L,), jnp.int32),
            pltpu.VMEM((L,), jnp.int32),
            pltpu.VMEM((TB, D), jnp.float32),
            pltpu.VMEM((TB, D), jnp.float32),
            pltpu.VMEM((TB, D), jnp.float32),
            pltpu.SemaphoreType.DMA,
            pltpu.SemaphoreType.DMA,
        ],
    )(yg, tok)

    return out.reshape(Bb, s, D)
